# baseline (speedup 1.0000x reference)
import functools

import jax
import jax.numpy as jnp
from jax import lax
from jax.experimental import pallas as pl
from jax.experimental.pallas import tpu as pltpu
from jax.experimental.pallas import tpu_sc as plsc

N = 10000
NE = 320000
D = 128

NC = 2
NS = 16
NW = NC * NS
EPW = NE // NW
K = 40
NCH = EPW // K
NACC = 10240
RPT = NACC // NS
K2 = 80
KS = 40
EPW2 = NE // NS
NCH2 = EPW2 // K2
NCHS = EPW2 // KS


def _pre_body(v_ref, ws_ref, wr_ref, wn_ref, os_ref, or_ref, on_ref):
    v = v_ref[...]
    os_ref[...] = jnp.dot(v, ws_ref[...], preferred_element_type=jnp.float32)
    or_ref[...] = jnp.dot(v, wr_ref[...], preferred_element_type=jnp.float32)
    on_ref[...] = jnp.dot(v, wn_ref[...], preferred_element_type=jnp.float32)


def _precompute(V2, W1e_s, W1e_r, W1n_v):
    blk = 1000
    return pl.pallas_call(
        _pre_body,
        grid=(N // blk,),
        in_specs=[
            pl.BlockSpec((blk, D), lambda i: (i, 0)),
            pl.BlockSpec((D, D), lambda i: (0, 0)),
            pl.BlockSpec((D, D), lambda i: (0, 0)),
            pl.BlockSpec((D, D), lambda i: (0, 0)),
        ],
        out_specs=[
            pl.BlockSpec((blk, D), lambda i: (i, 0)),
            pl.BlockSpec((blk, D), lambda i: (i, 0)),
            pl.BlockSpec((blk, D), lambda i: (i, 0)),
        ],
        out_shape=[jax.ShapeDtypeStruct((N, D), jnp.float32)] * 3,
    )(V2, W1e_s, W1e_r, W1n_v)


@functools.partial(
    pl.kernel,
    out_type=jax.ShapeDtypeStruct((2 * NE, D), jnp.float32),
    mesh=plsc.VectorSubcoreMesh(core_axis_name="c", subcore_axis_name="s",
                                num_cores=NC, num_subcores=NS),
    scratch_types=[
        [pltpu.VMEM((2 * K,), jnp.int32)] * 5,
        [pltpu.VMEM((2 * K, D), jnp.float32)] * 5,
        pltpu.SemaphoreType.DMA,
        pltpu.SemaphoreType.DMA,
    ],
)
def _sc_gather(vwc, idxc, g, iv, buf, sem, semw):
    wid = lax.axis_index("s") * NC + lax.axis_index("c")
    base = 2 * wid * EPW

    def group(p, carry):
        cps = []
        for t in range(5):
            off = base + (5 * p + t) * (2 * K)
            pltpu.sync_copy(idxc.at[pl.ds(off, 2 * K)], iv[t])
            cps.append(pltpu.async_copy(vwc.at[iv[t]], buf[t], sem))
        cpw = []
        for t in range(5):
            off = base + (5 * p + t) * (2 * K)
            cps[t].wait()
            cpw.append(pltpu.async_copy(buf[t], g.at[pl.ds(off, 2 * K)], semw))
        for t in range(5):
            cpw[t].wait()
        return carry

    lax.fori_loop(0, NCH // 5, group, 0)


def _edge_body(g_ref, e_ref, w1_ref, b1_ref, w2_ref, b2_ref, ee_ref):
    g = g_ref[...]
    h1 = (g[:, :D] + g[:, D:]
          + jnp.dot(e_ref[...], w1_ref[...], preferred_element_type=jnp.float32)
          + b1_ref[...])
    h = h1 * jax.nn.sigmoid(h1)
    ee_ref[...] = (jnp.dot(h, w2_ref[...], preferred_element_type=jnp.float32)
                   + b2_ref[...])


def _edge_mlp(G, E2, W1e_e, b1e, W2e, b2e):
    be = 2000
    return pl.pallas_call(
        _edge_body,
        grid=(NE // be,),
        in_specs=[
            pl.BlockSpec((be, 2 * D), lambda i: (i, 0)),
            pl.BlockSpec((be, D), lambda i: (i, 0)),
            pl.BlockSpec((D, D), lambda i: (0, 0)),
            pl.BlockSpec((1, D), lambda i: (0, 0)),
            pl.BlockSpec((D, D), lambda i: (0, 0)),
            pl.BlockSpec((1, D), lambda i: (0, 0)),
        ],
        out_specs=pl.BlockSpec((be, D), lambda i: (i, 0)),
        out_shape=jax.ShapeDtypeStruct((NE, D), jnp.float32),
    )(G, E2, W1e_e, b1e.reshape(1, D), W2e, b2e.reshape(1, D))


@functools.partial(
    pl.kernel,
    out_type=jax.ShapeDtypeStruct((NC, NACC, D), jnp.float32),
    mesh=plsc.VectorSubcoreMesh(core_axis_name="c", subcore_axis_name="s",
                                num_cores=NC, num_subcores=NS),
    scratch_types=[
        [pltpu.VMEM((KS,), jnp.int32)] * 5,
        [pltpu.VMEM((KS, D), jnp.float32)] * 5,
        pltpu.VMEM_SHARED((NACC, D), jnp.float32),
        pltpu.SemaphoreType.DMA,
    ],
)
def _sc_scatter(ee, idxcat, z_h, t_out, iv, ev, tt, sem):
    c = lax.axis_index("c")
    s = lax.axis_index("s")
    base = c * NE + s * EPW2
    row0 = s * RPT

    pltpu.sync_copy(z_h, ev[0])
    for q in range(RPT // KS):
        pltpu.sync_copy(ev[0], tt.at[pl.ds(row0 + q * KS, KS)])
    plsc.subcore_barrier()

    ones16 = jnp.full((16,), 1.0, jnp.float32)

    def group(p, carry):
        for t in range(5):
            off = base + (5 * p + t) * KS
            pltpu.sync_copy(idxcat.at[pl.ds(off, KS)], iv[t])
            pltpu.sync_copy(ee.at[pl.ds(off - c * NE, KS)], ev[t])
        col0 = (1 - c) * 64

        def fill(r, carry2):
            for t in range(5):
                for q in range(4):
                    ev[t][r, pl.ds(col0 + q * 16, 16)] = ones16
            return carry2

        lax.fori_loop(0, KS, fill, 0)
        cps = [pltpu.async_copy(ev[t], tt.at[iv[t]], sem, add=True)
               for t in range(5)]
        for t in range(5):
            cps[t].wait()
        return carry

    lax.fori_loop(0, NCHS // 5, group, 0)
    plsc.subcore_barrier()

    for q in range(RPT // KS):
        r = row0 + q * KS
        pltpu.sync_copy(tt.at[pl.ds(r, KS)], ev[0])
        pltpu.sync_copy(ev[0], t_out.at[c, pl.ds(r, KS)])


def _node_body(t_ref, vn_ref,
               w0_ref, w1_ref, b1_ref, w2_ref, b2_ref, out_ref):
    s0 = t_ref[0, :, 0:64]
    s1 = t_ref[1, :, 64:D]
    cnt0 = t_ref[0, :, 64:65]
    cnt1 = t_ref[1, :, 0:1]
    em0 = s0 / jnp.maximum(cnt0, 1.0)
    em1 = s1 / jnp.maximum(cnt1, 1.0)
    h1 = (vn_ref[...]
          + jnp.dot(em0, w0_ref[...], preferred_element_type=jnp.float32)
          + jnp.dot(em1, w1_ref[...], preferred_element_type=jnp.float32)
          + b1_ref[...])
    h = h1 * jax.nn.sigmoid(h1)
    out_ref[...] = (jnp.dot(h, w2_ref[...], preferred_element_type=jnp.float32)
                    + b2_ref[...])


def _node_mlp(T, VN, W1n_m0, W1n_m1, b1n, W2n, b2n):
    blk = 1000
    return pl.pallas_call(
        _node_body,
        grid=(N // blk,),
        in_specs=[
            pl.BlockSpec((NC, blk, D), lambda i: (0, i, 0)),
            pl.BlockSpec((blk, D), lambda i: (i, 0)),
            pl.BlockSpec((64, D), lambda i: (0, 0)),
            pl.BlockSpec((64, D), lambda i: (0, 0)),
            pl.BlockSpec((1, D), lambda i: (0, 0)),
            pl.BlockSpec((D, D), lambda i: (0, 0)),
            pl.BlockSpec((1, D), lambda i: (0, 0)),
        ],
        out_specs=pl.BlockSpec((blk, D), lambda i: (i, 0)),
        out_shape=jax.ShapeDtypeStruct((N, D), jnp.float32),
    )(T, VN, W1n_m0, W1n_m1,
      b1n.reshape(1, D), W2n, b2n.reshape(1, D))


def kernel(V, E, edges, W1e, b1e, W2e, b2e, W1n, b1n, W2n, b2n):
    V2 = V[0]
    E2 = E[0]
    idx0 = jnp.asarray(edges[0, :, 0], jnp.int32)
    idx1 = jnp.asarray(edges[0, :, 1], jnp.int32)
    idxc = jnp.stack([2 * idx0, 2 * idx1 + 1], axis=1).reshape(2 * NE)
    idxcat = jnp.concatenate([idx0, idx1])
    W1e_s, W1e_r, W1e_e = W1e[0:D], W1e[D:2 * D], W1e[2 * D:]
    W1n_v, W1n_m0, W1n_m1 = W1n[0:D], W1n[D:D + 64], W1n[D + 64:]

    VW_s, VW_r, VN = _precompute(V2, W1e_s, W1e_r, W1n_v)
    VWC = jnp.stack([VW_s, VW_r], axis=1).reshape(2 * N, D)
    G = _sc_gather(VWC, idxc)
    EE = _edge_mlp(G.reshape(NE, 2 * D), E2, W1e_e, b1e, W2e, b2e)

    z40_h = jnp.zeros((KS, D), jnp.float32)
    T = _sc_scatter(EE, idxcat, z40_h)

    NOut = _node_mlp(T, VN, W1n_m0, W1n_m1, b1n, W2n, b2n)
    return (NOut[None], EE[None])


# speedup vs baseline: 1.1584x; 1.1584x over previous
import functools

import jax
import jax.numpy as jnp
from jax import lax
from jax.experimental import pallas as pl
from jax.experimental.pallas import tpu as pltpu
from jax.experimental.pallas import tpu_sc as plsc

N = 10000
NE = 320000
D = 128

NC = 2
NS = 16
NW = NC * NS
EPW = NE // NW
K = 40
NCH = EPW // K
NACC = 10240
RPT = NACC // NS
KS = 80
EPW2 = NE // NS
NCHS = EPW2 // KS


def _pre_body(v_ref, ws_ref, wr_ref, wn_ref, os_ref, or_ref, on_ref):
    v = v_ref[...]
    os_ref[...] = jnp.dot(v, ws_ref[...], preferred_element_type=jnp.float32)
    or_ref[...] = jnp.dot(v, wr_ref[...], preferred_element_type=jnp.float32)
    on_ref[...] = jnp.dot(v, wn_ref[...], preferred_element_type=jnp.float32)


def _precompute(V2, W1e_s, W1e_r, W1n_v):
    blk = 1000
    return pl.pallas_call(
        _pre_body,
        grid=(N // blk,),
        in_specs=[
            pl.BlockSpec((blk, D), lambda i: (i, 0)),
            pl.BlockSpec((D, D), lambda i: (0, 0)),
            pl.BlockSpec((D, D), lambda i: (0, 0)),
            pl.BlockSpec((D, D), lambda i: (0, 0)),
        ],
        out_specs=[
            pl.BlockSpec((blk, D), lambda i: (i, 0)),
            pl.BlockSpec((blk, D), lambda i: (i, 0)),
            pl.BlockSpec((blk, D), lambda i: (i, 0)),
        ],
        out_shape=[jax.ShapeDtypeStruct((N, D), jnp.float32)] * 3,
    )(V2, W1e_s, W1e_r, W1n_v)


@functools.partial(
    pl.kernel,
    out_type=jax.ShapeDtypeStruct((2 * NE, D), jnp.float32),
    mesh=plsc.VectorSubcoreMesh(core_axis_name="c", subcore_axis_name="s",
                                num_cores=NC, num_subcores=NS),
    scratch_types=[
        [pltpu.VMEM((2 * K,), jnp.int32)] * 5,
        [pltpu.VMEM((2 * K, D), jnp.float32)] * 5,
        pltpu.SemaphoreType.DMA,
        pltpu.SemaphoreType.DMA,
    ],
)
def _sc_gather(vwc, idxc, g, iv, buf, sem, semw):
    wid = lax.axis_index("s") * NC + lax.axis_index("c")
    base = 2 * wid * EPW

    def group(p, carry):
        cps = []
        for t in range(5):
            off = base + (5 * p + t) * (2 * K)
            pltpu.sync_copy(idxc.at[pl.ds(off, 2 * K)], iv[t])
            cps.append(pltpu.async_copy(vwc.at[iv[t]], buf[t], sem))
        cpw = []
        for t in range(5):
            off = base + (5 * p + t) * (2 * K)
            cps[t].wait()
            cpw.append(pltpu.async_copy(buf[t], g.at[pl.ds(off, 2 * K)], semw))
        for t in range(5):
            cpw[t].wait()
        return carry

    lax.fori_loop(0, NCH // 5, group, 0)


def _edge_body(g_ref, e_ref, w1_ref, b1_ref, w2_ref, b2_ref, ee_ref):
    g = g_ref[...]
    h1 = (g[:, :D] + g[:, D:]
          + jnp.dot(e_ref[...], w1_ref[...], preferred_element_type=jnp.float32)
          + b1_ref[...])
    h = h1 * jax.nn.sigmoid(h1)
    ee_ref[...] = (jnp.dot(h, w2_ref[...], preferred_element_type=jnp.float32)
                   + b2_ref[...])


def _edge_mlp(G, E2, W1e_e, b1e, W2e, b2e):
    be = 2000
    return pl.pallas_call(
        _edge_body,
        grid=(NE // be,),
        in_specs=[
            pl.BlockSpec((be, 2 * D), lambda i: (i, 0)),
            pl.BlockSpec((be, D), lambda i: (i, 0)),
            pl.BlockSpec((D, D), lambda i: (0, 0)),
            pl.BlockSpec((1, D), lambda i: (0, 0)),
            pl.BlockSpec((D, D), lambda i: (0, 0)),
            pl.BlockSpec((1, D), lambda i: (0, 0)),
        ],
        out_specs=pl.BlockSpec((be, D), lambda i: (i, 0)),
        out_shape=jax.ShapeDtypeStruct((NE, D), jnp.float32),
    )(G, E2, W1e_e, b1e.reshape(1, D), W2e, b2e.reshape(1, D))


@functools.partial(
    pl.kernel,
    out_type=jax.ShapeDtypeStruct((NC, NACC, D), jnp.float32),
    mesh=plsc.VectorSubcoreMesh(core_axis_name="c", subcore_axis_name="s",
                                num_cores=NC, num_subcores=NS),
    scratch_types=[
        [pltpu.VMEM((KS,), jnp.int32)] * 2,
        [pltpu.VMEM((KS, D), jnp.float32)] * 2,
        pltpu.VMEM_SHARED((NACC, D), jnp.float32),
        pltpu.SemaphoreType.DMA,
    ],
)
def _sc_scatter(ee, idxcat, z_h, t_out, iv, ev, tt, sem):
    c = lax.axis_index("c")
    s = lax.axis_index("s")
    base = c * NE + s * EPW2
    row0 = s * RPT

    pltpu.sync_copy(z_h, ev[0])
    for q in range(RPT // KS):
        pltpu.sync_copy(ev[0], tt.at[pl.ds(row0 + q * KS, KS)])
    plsc.subcore_barrier()

    ones16 = jnp.full((16,), 1.0, jnp.float32)
    col0 = (1 - c) * 64

    def group(p, carry):
        for t in range(2):
            off = base + (2 * p + t) * KS
            pltpu.sync_copy(idxcat.at[pl.ds(off, KS)], iv[t])
            pltpu.sync_copy(ee.at[pl.ds(off - c * NE, KS)], ev[t])

        def fill(r, carry2):
            for t in range(2):
                ev[t][r, pl.ds(col0, 16)] = ones16
            return carry2

        lax.fori_loop(0, KS, fill, 0)
        cps = [pltpu.async_copy(ev[t], tt.at[iv[t]], sem, add=True)
               for t in range(2)]
        for t in range(2):
            cps[t].wait()
        return carry

    lax.fori_loop(0, NCHS // 2, group, 0)
    plsc.subcore_barrier()

    for q in range(RPT // KS):
        r = row0 + q * KS
        pltpu.sync_copy(tt.at[pl.ds(r, KS)], ev[0])
        pltpu.sync_copy(ev[0], t_out.at[c, pl.ds(r, KS)])


def _node_body(t_ref, vn_ref,
               w0_ref, w1_ref, b1_ref, w2_ref, b2_ref, out_ref):
    s0 = t_ref[0, :, 0:64]
    s1 = t_ref[1, :, 64:D]
    cnt0 = t_ref[0, :, 64:65]
    cnt1 = t_ref[1, :, 0:1]
    em0 = s0 / jnp.maximum(cnt0, 1.0)
    em1 = s1 / jnp.maximum(cnt1, 1.0)
    h1 = (vn_ref[...]
          + jnp.dot(em0, w0_ref[...], preferred_element_type=jnp.float32)
          + jnp.dot(em1, w1_ref[...], preferred_element_type=jnp.float32)
          + b1_ref[...])
    h = h1 * jax.nn.sigmoid(h1)
    out_ref[...] = (jnp.dot(h, w2_ref[...], preferred_element_type=jnp.float32)
                    + b2_ref[...])


def _node_mlp(T, VN, W1n_m0, W1n_m1, b1n, W2n, b2n):
    blk = 1000
    return pl.pallas_call(
        _node_body,
        grid=(N // blk,),
        in_specs=[
            pl.BlockSpec((NC, blk, D), lambda i: (0, i, 0)),
            pl.BlockSpec((blk, D), lambda i: (i, 0)),
            pl.BlockSpec((64, D), lambda i: (0, 0)),
            pl.BlockSpec((64, D), lambda i: (0, 0)),
            pl.BlockSpec((1, D), lambda i: (0, 0)),
            pl.BlockSpec((D, D), lambda i: (0, 0)),
            pl.BlockSpec((1, D), lambda i: (0, 0)),
        ],
        out_specs=pl.BlockSpec((blk, D), lambda i: (i, 0)),
        out_shape=jax.ShapeDtypeStruct((N, D), jnp.float32),
    )(T, VN, W1n_m0, W1n_m1,
      b1n.reshape(1, D), W2n, b2n.reshape(1, D))


def kernel(V, E, edges, W1e, b1e, W2e, b2e, W1n, b1n, W2n, b2n):
    V2 = V[0]
    E2 = E[0]
    idx0 = jnp.asarray(edges[0, :, 0], jnp.int32)
    idx1 = jnp.asarray(edges[0, :, 1], jnp.int32)
    idxc = jnp.stack([2 * idx0, 2 * idx1 + 1], axis=1).reshape(2 * NE)
    idxcat = jnp.concatenate([idx0, idx1])
    W1e_s, W1e_r, W1e_e = W1e[0:D], W1e[D:2 * D], W1e[2 * D:]
    W1n_v, W1n_m0, W1n_m1 = W1n[0:D], W1n[D:D + 64], W1n[D + 64:]

    VW_s, VW_r, VN = _precompute(V2, W1e_s, W1e_r, W1n_v)
    VWC = jnp.stack([VW_s, VW_r], axis=1).reshape(2 * N, D)
    G = _sc_gather(VWC, idxc)
    EE = _edge_mlp(G.reshape(NE, 2 * D), E2, W1e_e, b1e, W2e, b2e)

    z40_h = jnp.zeros((KS, D), jnp.float32)
    T = _sc_scatter(EE, idxcat, z40_h)

    NOut = _node_mlp(T, VN, W1n_m0, W1n_m1, b1n, W2n, b2n)
    return (NOut[None], EE[None])
